# Initial kernel scaffold; baseline (speedup 1.0000x reference)
import jax
import jax.numpy as jnp
from jax.experimental import pallas as pl


def kernel(gene_ids, table):
    raise NotImplementedError("write your pallas kernel here")


# speedup vs baseline: 3.5430x; 3.5430x over previous
import functools

import jax
import jax.numpy as jnp
from jax import lax
from jax.experimental import pallas as pl
from jax.experimental.pallas import tpu as pltpu
from jax.experimental.pallas import tpu_sc as plsc

_NC = 2
_NS = 16
_NW = _NC * _NS
_CHUNK = 128


def _gather_body(n_chunks, dim, table_hbm, idx_hbm, out_hbm, idx_v, rows_v, sem):
    wid = lax.axis_index("s") * _NC + lax.axis_index("c")
    base = wid * (n_chunks * _CHUNK)
    pltpu.sync_copy(idx_hbm.at[wid], idx_v)

    def body(j, carry):
        pltpu.async_copy(table_hbm.at[idx_v.at[j]], rows_v, sem).wait()
        pltpu.sync_copy(rows_v, out_hbm.at[pl.ds(base + j * _CHUNK, _CHUNK)])
        return carry

    lax.fori_loop(0, n_chunks, body, 0, unroll=False)


@functools.partial(jax.jit, static_argnames=("n_chunks", "dim"))
def _gather(table, idx3, *, n_chunks, dim):
    mesh = plsc.VectorSubcoreMesh(core_axis_name="c", subcore_axis_name="s")
    total = _NW * n_chunks * _CHUNK
    run = pl.kernel(
        functools.partial(_gather_body, n_chunks, dim),
        out_type=jax.ShapeDtypeStruct((total, dim), jnp.float32),
        mesh=mesh,
        scratch_types=[
            pltpu.VMEM((n_chunks, _CHUNK), jnp.int32),
            pltpu.VMEM((_CHUNK, dim), jnp.float32),
            pltpu.SemaphoreType.DMA,
        ],
        compiler_params=pltpu.CompilerParams(use_tc_tiling_on_sc=False),
    )
    return run(table, idx3)


def kernel(gene_ids, table):
    batch, seq = gene_ids.shape
    dim = table.shape[1]
    total = batch * seq
    assert total % (_NW * _CHUNK) == 0
    n_chunks = total // (_NW * _CHUNK)
    idx3 = gene_ids.reshape(_NW, n_chunks, _CHUNK)
    out = _gather(table, idx3, n_chunks=n_chunks, dim=dim)
    return out.reshape(batch, seq, dim)


# speedup vs baseline: 4.2664x; 1.2042x over previous
import functools

import jax
import jax.numpy as jnp
from jax import lax
from jax.experimental import pallas as pl
from jax.experimental.pallas import tpu as pltpu
from jax.experimental.pallas import tpu_sc as plsc

_NC = 2
_NS = 16
_NW = _NC * _NS
_CHUNK = 128
_K = 4
_GROUP = _K * _CHUNK


def _gather_body(n_groups, dim, table_hbm, idx_hbm, out_hbm, idx_v, buf_v, gsem, osem):
    wid = lax.axis_index("s") * _NC + lax.axis_index("c")
    base = wid * (n_groups * _GROUP)
    pltpu.sync_copy(idx_hbm.at[wid], idx_v)

    def gather_descs(g, s):
        return [
            pltpu.make_async_copy(
                table_hbm.at[idx_v.at[g * _K + k]],
                buf_v.at[s, pl.ds(k * _CHUNK, _CHUNK), :],
                gsem.at[s],
            )
            for k in range(_K)
        ]

    def write_desc(g, s):
        return pltpu.make_async_copy(
            buf_v.at[s],
            out_hbm.at[pl.ds(base + g * _GROUP, _GROUP)],
            osem.at[s],
        )

    for d in gather_descs(0, 0):
        d.start()

    @pl.loop(0, n_groups, step=2)
    def _(g0):
        for h in (0, 1):
            g = g0 + h
            s = h
            sn = 1 - h

            @pl.when(g + 1 < n_groups)
            def _():
                @pl.when(g >= 1)
                def _():
                    write_desc(g - 1, sn).wait()

                for d in gather_descs(g + 1, sn):
                    d.start()

            for d in gather_descs(g, s):
                d.wait()
            write_desc(g, s).start()

    write_desc(n_groups - 2, (n_groups - 2) % 2).wait()
    write_desc(n_groups - 1, (n_groups - 1) % 2).wait()


@functools.partial(jax.jit, static_argnames=("n_groups", "dim"))
def _gather(table, idx3, *, n_groups, dim):
    mesh = plsc.VectorSubcoreMesh(core_axis_name="c", subcore_axis_name="s")
    total = _NW * n_groups * _GROUP
    run = pl.kernel(
        functools.partial(_gather_body, n_groups, dim),
        out_type=jax.ShapeDtypeStruct((total, dim), jnp.float32),
        mesh=mesh,
        scratch_types=[
            pltpu.VMEM((n_groups * _K, _CHUNK), jnp.int32),
            pltpu.VMEM((2, _GROUP, dim), jnp.float32),
            pltpu.SemaphoreType.DMA((2,)),
            pltpu.SemaphoreType.DMA((2,)),
        ],
        compiler_params=pltpu.CompilerParams(use_tc_tiling_on_sc=False),
    )
    return run(table, idx3)


def kernel(gene_ids, table):
    batch, seq = gene_ids.shape
    dim = table.shape[1]
    total = batch * seq
    assert total % (_NW * _GROUP) == 0
    n_groups = total // (_NW * _GROUP)
    assert n_groups % 2 == 0
    idx3 = gene_ids.reshape(_NW, n_groups * _K, _CHUNK)
    out = _gather(table, idx3, n_groups=n_groups, dim=dim)
    return out.reshape(batch, seq, dim)
